# baseline (speedup 1.0000x reference)
import functools

import jax
import jax.numpy as jnp
from jax.experimental import pallas as pl
from jax.experimental.pallas import tpu as pltpu


def _mish(x):
    e = jnp.exp(jnp.minimum(x, 20.0))
    g = e * (e + 2.0)
    return x * (g / (g + 2.0))


def _impala_kernel(a_ref, b_ref, wc_ref, bc_ref, w11_ref, b11_ref,
                   w12_ref, b12_ref, w21_ref, b21_ref, w22_ref, b22_ref,
                   out_ref, im1_ref, cop_ref, xm_ref, ap_ref, bp_ref, im2_ref,
                   *, H, W, Cin, C):
    Ho, Wo = H // 2, W // 2
    P = H * Wo

    for dy in range(3):
        for dx in range(3):
            t = dy * 3 + dx
            e_src, o_src = (
                (a_ref[dy:dy + H, 0:Wo, :], b_ref[dy:dy + H, 0:Wo, :]),
                (b_ref[dy:dy + H, 0:Wo, :], a_ref[dy:dy + H, 1:Wo + 1, :]),
                (a_ref[dy:dy + H, 1:Wo + 1, :], b_ref[dy:dy + H, 1:Wo + 1, :]),
            )[dx]
            im1_ref[0:P, t * Cin:(t + 1) * Cin] = e_src.reshape(P, Cin)
            im1_ref[P:2 * P, t * Cin:(t + 1) * Cin] = o_src.reshape(P, Cin)
    c = jnp.dot(im1_ref[...], wc_ref[...], preferred_element_type=jnp.float32)
    c = _mish(c + bc_ref[...])
    ce = c[0:P].reshape(H, Wo, C)
    cop_ref[:, 0:1, :] = jnp.full((H, 1, C), -jnp.inf, jnp.float32)
    cop_ref[:, 1:Wo + 1, :] = c[P:2 * P].reshape(H, Wo, C)

    xm_ref[0:1] = jnp.full((1, Wo, C), -jnp.inf, jnp.float32)
    xm_ref[H + 1:H + 2] = jnp.full((1, Wo, C), -jnp.inf, jnp.float32)
    xm_ref[1:H + 1] = jnp.maximum(ce, jnp.maximum(cop_ref[:, 1:Wo + 1, :],
                                                  cop_ref[:, 0:Wo, :]))
    ym = jnp.maximum(xm_ref[0:H], jnp.maximum(xm_ref[1:H + 1],
                                              xm_ref[2:H + 2]))
    m = ym.reshape(Ho, 2, Wo, C)[:, 0]

    def conv_mish(src_ref, w_ref, b_ref):
        for dy in range(3):
            for dx in range(3):
                t = dy * 3 + dx
                im2_ref[:, t * C:(t + 1) * C] = (
                    src_ref[dy:dy + Ho, dx:dx + Wo, :].reshape(Ho * Wo, C))
        y = jnp.dot(im2_ref[...], w_ref[...],
                    preferred_element_type=jnp.float32)
        return _mish(y + b_ref[...])

    ap_ref[...] = jnp.zeros((Ho + 2, Wo + 2, C), jnp.float32)
    ap_ref[1:Ho + 1, 1:Wo + 1, :] = m

    for w1_ref, b1_ref, w2_ref, b2_ref in (
            (w11_ref, b11_ref, w12_ref, b12_ref),
            (w21_ref, b21_ref, w22_ref, b22_ref)):
        h = conv_mish(ap_ref, w1_ref, b1_ref)
        bp_ref[...] = jnp.zeros((Ho + 2, Wo + 2, C), jnp.float32)
        bp_ref[1:Ho + 1, 1:Wo + 1, :] = h.reshape(Ho, Wo, C)
        y = conv_mish(bp_ref, w2_ref, b2_ref)
        y = y + ap_ref[1:Ho + 1, 1:Wo + 1, :].reshape(Ho * Wo, C)
        ap_ref[1:Ho + 1, 1:Wo + 1, :] = y.reshape(Ho, Wo, C)

    out_ref[...] = ap_ref[1:Ho + 1, 1:Wo + 1, :]


def kernel(x, conv_w, conv_b, res1_w1, res1_b1, res1_w2, res1_b2,
           res2_w1, res2_b1, res2_w2, res2_b2):
    n, cin, h, w = x.shape
    cout = conv_w.shape[-1]
    ho, wo = h // 2, w // 2

    xh = jnp.transpose(x, (0, 2, 3, 1))
    xp = jnp.pad(xh, ((0, 0), (1, 1), (1, 1), (0, 0)))
    a = xp[:, :, 0::2, :]
    b = xp[:, :, 1::2, :]

    wc = conv_w.reshape(9 * cin, cout)
    ws = [m.reshape(9 * cout, cout)
          for m in (res1_w1, res1_w2, res2_w1, res2_w2)]
    bs = [v.reshape(1, cout)
          for v in (conv_b, res1_b1, res1_b2, res2_b1, res2_b2)]

    kern = functools.partial(_impala_kernel, H=h, W=w, Cin=cin, C=cout)
    xspec = pl.BlockSpec((None, h + 2, wo + 1, cin), lambda i: (i, 0, 0, 0))
    wspec = pl.BlockSpec((9 * cout, cout), lambda i: (0, 0))
    bspec = pl.BlockSpec((1, cout), lambda i: (0, 0))
    out = pl.pallas_call(
        kern,
        grid=(n,),
        in_specs=[
            xspec, xspec,
            pl.BlockSpec((9 * cin, cout), lambda i: (0, 0)), bspec,
            wspec, bspec, wspec, bspec,
            wspec, bspec, wspec, bspec,
        ],
        out_specs=pl.BlockSpec((None, ho, wo, cout), lambda i: (i, 0, 0, 0)),
        out_shape=jax.ShapeDtypeStruct((n, ho, wo, cout), jnp.float32),
        scratch_shapes=[
            pltpu.VMEM((h * w, 9 * cin), jnp.float32),
            pltpu.VMEM((h, wo + 1, cout), jnp.float32),
            pltpu.VMEM((h + 2, wo, cout), jnp.float32),
            pltpu.VMEM((ho + 2, wo + 2, cout), jnp.float32),
            pltpu.VMEM((ho + 2, wo + 2, cout), jnp.float32),
            pltpu.VMEM((ho * wo, 9 * cout), jnp.float32),
        ],
        compiler_params=pltpu.CompilerParams(
            dimension_semantics=("parallel",),
            vmem_limit_bytes=64 * 1024 * 1024),
        cost_estimate=pl.CostEstimate(
            flops=2 * n * (h * w * 9 * cin * cout
                           + 4 * ho * wo * 9 * cout * cout),
            transcendentals=n * (h * w + 4 * ho * wo) * cout,
            bytes_accessed=4 * (2 * n * (h + 2) * (wo + 1) * cin
                                + n * ho * wo * cout),
        ),
    )(a, b, wc, bs[0], ws[0], bs[1], ws[1], bs[2], ws[2], bs[3], ws[3], bs[4])
    return jnp.transpose(out, (0, 3, 1, 2))


# speedup vs baseline: 6.1988x; 6.1988x over previous
import functools

import jax
import jax.numpy as jnp
from jax.experimental import pallas as pl
from jax.experimental.pallas import tpu as pltpu

_S = 8


def _mish(x):
    e = jnp.exp(jnp.minimum(x, 20.0))
    g = e * (e + 2.0)
    return x * (g / (g + 2.0))


def _impala_kernel(a_ref, b_ref, wc_ref, bc_ref, w11_ref, b11_ref,
                   w12_ref, b12_ref, w21_ref, b21_ref, w22_ref, b22_ref,
                   out_ref, im1_ref, cop_ref, xm_ref, ap_ref, bp_ref, im2_ref,
                   *, H, W, Cin, C):
    Ho, Wo = H // 2, W // 2
    P = H * Wo
    Li, Lo = _S * Cin, _S * C

    for dy in range(3):
        for dx in range(3):
            t = dy * 3 + dx
            e_src, o_src = (
                (a_ref[dy:dy + H, 0:Wo, :], b_ref[dy:dy + H, 0:Wo, :]),
                (b_ref[dy:dy + H, 0:Wo, :], a_ref[dy:dy + H, 1:Wo + 1, :]),
                (a_ref[dy:dy + H, 1:Wo + 1, :], b_ref[dy:dy + H, 1:Wo + 1, :]),
            )[dx]
            im1_ref[0:P, t * Li:(t + 1) * Li] = e_src.reshape(P, Li)
            im1_ref[P:2 * P, t * Li:(t + 1) * Li] = o_src.reshape(P, Li)
    c = jnp.dot(im1_ref[...], wc_ref[...], preferred_element_type=jnp.float32)
    c = _mish(c + bc_ref[...])
    ce = c[0:P].reshape(H, Wo, Lo)
    cop_ref[:, 0:1, :] = jnp.full((H, 1, Lo), -jnp.inf, jnp.float32)
    cop_ref[:, 1:Wo + 1, :] = c[P:2 * P].reshape(H, Wo, Lo)

    xm_ref[0:1] = jnp.full((1, Wo, Lo), -jnp.inf, jnp.float32)
    xm_ref[H + 1:H + 2] = jnp.full((1, Wo, Lo), -jnp.inf, jnp.float32)
    xm_ref[1:H + 1] = jnp.maximum(ce, jnp.maximum(cop_ref[:, 1:Wo + 1, :],
                                                  cop_ref[:, 0:Wo, :]))
    ym = jnp.maximum(xm_ref[0:H], jnp.maximum(xm_ref[1:H + 1],
                                              xm_ref[2:H + 2]))
    m = ym.reshape(Ho, 2, Wo, Lo)[:, 0]

    def conv_mish(src_ref, w_ref, b_ref):
        for dy in range(3):
            for dx in range(3):
                t = dy * 3 + dx
                im2_ref[:, t * Lo:(t + 1) * Lo] = (
                    src_ref[dy:dy + Ho, dx:dx + Wo, :].reshape(Ho * Wo, Lo))
        y = jnp.dot(im2_ref[...], w_ref[...],
                    preferred_element_type=jnp.float32)
        return _mish(y + b_ref[...])

    ap_ref[...] = jnp.zeros((Ho + 2, Wo + 2, Lo), jnp.float32)
    ap_ref[1:Ho + 1, 1:Wo + 1, :] = m

    for w1_ref, b1_ref, w2_ref, b2_ref in (
            (w11_ref, b11_ref, w12_ref, b12_ref),
            (w21_ref, b21_ref, w22_ref, b22_ref)):
        h = conv_mish(ap_ref, w1_ref, b1_ref)
        bp_ref[...] = jnp.zeros((Ho + 2, Wo + 2, Lo), jnp.float32)
        bp_ref[1:Ho + 1, 1:Wo + 1, :] = h.reshape(Ho, Wo, Lo)
        y = conv_mish(bp_ref, w2_ref, b2_ref)
        y = y + ap_ref[1:Ho + 1, 1:Wo + 1, :].reshape(Ho * Wo, Lo)
        ap_ref[1:Ho + 1, 1:Wo + 1, :] = y.reshape(Ho, Wo, Lo)

    out_ref[...] = ap_ref[1:Ho + 1, 1:Wo + 1, :]


def _block_diag_w(w, ci, co):
    eye = jnp.eye(_S, dtype=w.dtype)
    wb = jnp.einsum('su,tco->tscuo', eye, w.reshape(9, ci, co))
    return wb.reshape(9 * _S * ci, _S * co)


def kernel(x, conv_w, conv_b, res1_w1, res1_b1, res1_w2, res1_b2,
           res2_w1, res2_b1, res2_w2, res2_b2):
    n, cin, h, w = x.shape
    cout = conv_w.shape[-1]
    ho, wo = h // 2, w // 2
    g = n // _S

    xg = jnp.transpose(x.reshape(g, _S, cin, h, w), (0, 3, 4, 1, 2))
    xg = xg.reshape(g, h, w, _S * cin)
    xp = jnp.pad(xg, ((0, 0), (1, 1), (1, 1), (0, 0)))
    a = xp[:, :, 0::2, :]
    b = xp[:, :, 1::2, :]

    wc = _block_diag_w(conv_w, cin, cout)
    ws = [_block_diag_w(m, cout, cout)
          for m in (res1_w1, res1_w2, res2_w1, res2_w2)]
    bs = [jnp.tile(v.reshape(1, cout), (1, _S))
          for v in (conv_b, res1_b1, res1_b2, res2_b1, res2_b2)]

    li, lo = _S * cin, _S * cout
    kern = functools.partial(_impala_kernel, H=h, W=w, Cin=cin, C=cout)
    xspec = pl.BlockSpec((None, h + 2, wo + 1, li), lambda i: (i, 0, 0, 0))
    wcspec = pl.BlockSpec((9 * li, lo), lambda i: (0, 0))
    wspec = pl.BlockSpec((9 * lo, lo), lambda i: (0, 0))
    bspec = pl.BlockSpec((1, lo), lambda i: (0, 0))
    out = pl.pallas_call(
        kern,
        grid=(g,),
        in_specs=[
            xspec, xspec,
            wcspec, bspec,
            wspec, bspec, wspec, bspec,
            wspec, bspec, wspec, bspec,
        ],
        out_specs=pl.BlockSpec((None, ho, wo, lo), lambda i: (i, 0, 0, 0)),
        out_shape=jax.ShapeDtypeStruct((g, ho, wo, lo), jnp.float32),
        scratch_shapes=[
            pltpu.VMEM((h * w, 9 * li), jnp.float32),
            pltpu.VMEM((h, wo + 1, lo), jnp.float32),
            pltpu.VMEM((h + 2, wo, lo), jnp.float32),
            pltpu.VMEM((ho + 2, wo + 2, lo), jnp.float32),
            pltpu.VMEM((ho + 2, wo + 2, lo), jnp.float32),
            pltpu.VMEM((ho * wo, 9 * lo), jnp.float32),
        ],
        compiler_params=pltpu.CompilerParams(
            dimension_semantics=("parallel",),
            vmem_limit_bytes=100 * 1024 * 1024),
        cost_estimate=pl.CostEstimate(
            flops=2 * n * (h * w * 9 * cin * cout
                           + 4 * ho * wo * 9 * cout * cout),
            transcendentals=n * (h * w + 4 * ho * wo) * cout,
            bytes_accessed=4 * (2 * g * (h + 2) * (wo + 1) * li
                                + g * ho * wo * lo),
        ),
    )(a, b, wc, bs[0], ws[0], bs[1], ws[1], bs[2], ws[2], bs[3], ws[3], bs[4])
    out = out.reshape(g, ho, wo, _S, cout)
    return jnp.transpose(out, (0, 3, 4, 1, 2)).reshape(n, cout, ho, wo)
